# baseline (speedup 1.0000x reference)
import functools

import jax
import jax.numpy as jnp
import numpy as np
from jax import lax
from jax.experimental import pallas as pl
from jax.experimental.pallas import tpu as pltpu
from jax.experimental.pallas import tpu_sc as plsc

_K = 64
_SLACK = 80


def _fps_pallas(pos, n_samples):
    N = pos.shape[0]
    SR, LC = N // 128, 128
    OR = n_samples // 128

    def body(px_ref, py_ref, pz_ref, sel_ref, qx_ref, qy_ref, qz_ref):
        iota_src = (lax.broadcasted_iota(jnp.int32, (SR, LC), 0) * 128
                    + lax.broadcasted_iota(jnp.int32, (SR, LC), 1))
        iota_sel = (lax.broadcasted_iota(jnp.int32, (OR, LC), 0) * 128
                    + lax.broadcasted_iota(jnp.int32, (OR, LC), 1))
        px, py, pz = px_ref[...], py_ref[...], pz_ref[...]
        lx0, ly0, lz0 = px[0, 0], py[0, 0], pz[0, 0]
        sel_ref[...] = jnp.zeros((OR, LC), jnp.int32)
        qx_ref[...] = jnp.where(iota_sel == 0, lx0, 0.0)
        qy_ref[...] = jnp.where(iota_sel == 0, ly0, 0.0)
        qz_ref[...] = jnp.where(iota_sel == 0, lz0, 0.0)
        dists0 = jnp.full((SR, LC), jnp.inf, jnp.float32)

        def step(i, carry):
            lx, ly, lz, dists = carry
            dx, dy, dz = px - lx, py - ly, pz - lz
            d = (dx * dx + dy * dy) + dz * dz
            dists = jnp.minimum(dists, d)
            m = jnp.max(dists)
            idx = jnp.min(jnp.where(dists == m, iota_src, jnp.int32(2**30)))
            hit = iota_src == idx
            nlx = jnp.max(jnp.where(hit, px, -1.0))
            nly = jnp.max(jnp.where(hit, py, -1.0))
            nlz = jnp.max(jnp.where(hit, pz, -1.0))
            put = iota_sel == i
            sel_ref[...] = jnp.where(put, idx, sel_ref[...])
            qx_ref[...] = jnp.where(put, nlx, qx_ref[...])
            qy_ref[...] = jnp.where(put, nly, qy_ref[...])
            qz_ref[...] = jnp.where(put, nlz, qz_ref[...])
            return (nlx, nly, nlz, dists)

        lax.fori_loop(1, n_samples, step, (lx0, ly0, lz0, dists0))

    px = pos[:, 0].reshape(SR, LC)
    py = pos[:, 1].reshape(SR, LC)
    pz = pos[:, 2].reshape(SR, LC)
    sel, qx, qy, qz = pl.pallas_call(
        body,
        out_shape=(jax.ShapeDtypeStruct((OR, LC), jnp.int32),
                   jax.ShapeDtypeStruct((OR, LC), jnp.float32),
                   jax.ShapeDtypeStruct((OR, LC), jnp.float32),
                   jax.ShapeDtypeStruct((OR, LC), jnp.float32)),
    )(px, py, pz)
    pos_sel = jnp.stack([qx.reshape(-1), qy.reshape(-1), qz.reshape(-1)], axis=1)
    return sel.reshape(-1), pos_sel


def _d2_pallas(pos_x, pos_y):
    Q, N = pos_y.shape[0], pos_x.shape[0]
    BQ = min(256, Q)

    def body(y_ref, xt_ref, sy_ref, sx_ref, o_ref):
        t = jnp.dot(y_ref[...], xt_ref[...])
        o_ref[...] = (sy_ref[...].reshape(BQ, 1)
                      + sx_ref[...].reshape(1, N)) - 2.0 * t

    sy = jnp.sum(pos_y ** 2, axis=1)
    sx = jnp.sum(pos_x ** 2, axis=1)
    return pl.pallas_call(
        body,
        grid=(Q // BQ,),
        in_specs=[
            pl.BlockSpec((BQ, 3), lambda i: (i, 0)),
            pl.BlockSpec((3, N), lambda i: (0, 0)),
            pl.BlockSpec((BQ,), lambda i: (i,)),
            pl.BlockSpec((N,), lambda i: (0,)),
        ],
        out_specs=pl.BlockSpec((BQ, N), lambda i: (i, 0)),
        out_shape=jax.ShapeDtypeStruct((Q, N), jnp.float32),
    )(pos_y, pos_x.T, sy, sx)


def _select_gather_sc(d2_flat, table, r2, Q, N, F):
    NW = 32
    QW = Q // NW
    NB = N // 16
    mesh = plsc.VectorSubcoreMesh(core_axis_name="c", subcore_axis_name="s")

    @functools.partial(
        pl.kernel, mesh=mesh,
        compiler_params=pltpu.CompilerParams(
            needs_layout_passes=False, use_tc_tiling_on_sc=False),
        out_type=(jax.ShapeDtypeStruct((Q * _K, F), jnp.float32),
                  jax.ShapeDtypeStruct((Q,), jnp.int32)),
        scratch_types=[
            pltpu.VMEM((N,), jnp.float32),
            pltpu.VMEM((QW * _SLACK,), jnp.int32),
            pltpu.VMEM((_K, F), jnp.float32),
            pltpu.VMEM((QW,), jnp.int32),
            pltpu.SemaphoreType.DMA,
        ],
    )
    def k(d2_hbm, tab_hbm, g_hbm, cnt_hbm, row_v, nbr_v, rows_v, cnt_v, sem):
        wid = lax.axis_index("s") * 2 + lax.axis_index("c")
        base = wid * QW
        lane = lax.iota(jnp.int32, 16)
        lane0 = lane == 0

        def per_query(ql, carry):
            q = base + ql
            pltpu.sync_copy(d2_hbm.at[pl.ds(q * N, N)], row_v)
            for off in range(0, _SLACK, 16):
                nbr_v[pl.ds(ql * _SLACK + off, 16)] = jnp.zeros((16,), jnp.int32)

            def scan(jb, cnt):
                dv = row_v[pl.ds(jb * 16, 16)]
                m = dv <= r2
                jv = lane + jb * 16
                rank = jnp.cumsum(m.astype(jnp.int32))
                slot = cnt + rank - 1
                sel = jnp.logical_and(m, slot < _K)
                plsc.store_scatter(nbr_v, [ql * _SLACK + slot], jv, mask=sel)
                pc = plsc.all_reduce_population_count(m)
                return cnt + pc[0]

            cnt = lax.fori_loop(0, NB, scan, jnp.int32(0))
            cntq = jnp.minimum(cnt, jnp.int32(_K))
            plsc.store_scatter(cnt_v, [jnp.broadcast_to(ql, (16,))],
                               jnp.broadcast_to(cntq, (16,)), mask=lane0)
            pltpu.async_copy(
                tab_hbm.at[nbr_v.at[pl.ds(ql * _SLACK, _K)]], rows_v, sem
            ).wait()
            pltpu.sync_copy(rows_v, g_hbm.at[pl.ds(q * _K, _K)])
            return carry

        lax.fori_loop(0, QW, per_query, 0)
        pltpu.sync_copy(cnt_v, cnt_hbm.at[pl.ds(base, QW)])

    return k(d2_flat, table)


def _conv_pallas(G3, posrep, vmask, xdim, W1, b1, W2, b2):
    Q, _, F = G3.shape
    H = W1.shape[1]
    C = W2.shape[1]
    BQ = min(128, Q)

    def body(g_ref, pr_ref, vm_ref, w1_ref, b1_ref, w2_ref, b2_ref, o_ref):
        g2 = g_ref[...].reshape(BQ * _K, F)
        p2 = pr_ref[...].reshape(BQ * _K, 4)
        rel = g2[:, xdim:xdim + 3] - p2[:, :3]
        cat = jnp.concatenate([g2[:, :xdim], rel], axis=1)
        h = jnp.maximum(jnp.dot(cat, w1_ref[...]) + b1_ref[...], 0.0)
        msg = jnp.dot(h, w2_ref[...]) + b2_ref[...]
        m3 = msg.reshape(BQ, _K, C)
        m3 = jnp.where(vm_ref[...] > 0, m3, -jnp.inf)
        o_ref[...] = jnp.max(m3, axis=1)

    return pl.pallas_call(
        body,
        grid=(Q // BQ,),
        in_specs=[
            pl.BlockSpec((BQ, _K, F), lambda i: (i, 0, 0)),
            pl.BlockSpec((BQ, _K, 4), lambda i: (i, 0, 0)),
            pl.BlockSpec((BQ, _K, 1), lambda i: (i, 0, 0)),
            pl.BlockSpec((xdim + 3, H), lambda i: (0, 0)),
            pl.BlockSpec((1, H), lambda i: (0, 0)),
            pl.BlockSpec((H, C), lambda i: (0, 0)),
            pl.BlockSpec((1, C), lambda i: (0, 0)),
        ],
        out_specs=pl.BlockSpec((BQ, C), lambda i: (i, 0)),
        out_shape=jax.ShapeDtypeStruct((Q, C), jnp.float32),
    )(G3, posrep, vmask, W1, b1, W2, b2)


def _sa_stage(pos_src, x_src, pos_dst, r, W1, b1, W2, b2):
    N = pos_src.shape[0]
    Q = pos_dst.shape[0]
    xdim = 3 if x_src is None else x_src.shape[1]
    F = ((xdim + 3 + 15) // 16) * 16

    parts = [pos_src if x_src is None else x_src, pos_src]
    table = jnp.concatenate(parts, axis=1)
    table = jnp.pad(table, ((0, 0), (0, F - (xdim + 3))))

    d2 = _d2_pallas(pos_src, pos_dst)
    r2 = np.float32(r * r)
    G, cnt = _select_gather_sc(d2.reshape(-1), table, r2, Q, N, F)
    G3 = G.reshape(Q, _K, F)

    posrep = jnp.broadcast_to(
        jnp.pad(pos_dst, ((0, 0), (0, 1)))[:, None, :], (Q, _K, 4))
    vmask = (jnp.arange(_K, dtype=jnp.int32)[None, :]
             < cnt[:, None]).astype(jnp.float32)[:, :, None]
    return _conv_pallas(G3, posrep, vmask, xdim, W1,
                        b1.reshape(1, -1), W2, b2.reshape(1, -1))


def kernel(pos, W11, b11, W12, b12, W21, b21, W22, b22, W31, b31, W32, b32,
           batch):
    n1 = pos.shape[0] // 2
    idx1, pos1 = _fps_pallas(pos, n1)
    x1 = _sa_stage(pos, None, pos1, 0.2, W11, b11, W12, b12)
    n2 = n1 // 4
    idx2, pos2 = _fps_pallas(pos1, n2)
    x2 = _sa_stage(pos1, x1, pos2, 0.4, W21, b21, W22, b22)
    x3 = _sa_stage(pos2, x2, pos2, 1.0, W31, b31, W32, b32)
    batch3 = jnp.take(jnp.take(batch, idx1), idx2)
    return (x3, pos2, batch3)


# speedup vs baseline: 1.1801x; 1.1801x over previous
import functools

import jax
import jax.numpy as jnp
import numpy as np
from jax import lax
from jax.experimental import pallas as pl
from jax.experimental.pallas import tpu as pltpu
from jax.experimental.pallas import tpu_sc as plsc

_K = 64
_SLACK = 80


def _fps_pallas(pos, n_samples):
    N = pos.shape[0]
    SR, LC = N // 128, 128
    OR = n_samples // 128

    def body(px_ref, py_ref, pz_ref, sel_ref, qx_ref, qy_ref, qz_ref):
        iota_src = (lax.broadcasted_iota(jnp.int32, (SR, LC), 0) * 128
                    + lax.broadcasted_iota(jnp.int32, (SR, LC), 1))
        iota_sel = (lax.broadcasted_iota(jnp.int32, (OR, LC), 0) * 128
                    + lax.broadcasted_iota(jnp.int32, (OR, LC), 1))
        px, py, pz = px_ref[...], py_ref[...], pz_ref[...]
        lx0, ly0, lz0 = px[0, 0], py[0, 0], pz[0, 0]
        sel_ref[...] = jnp.zeros((OR, LC), jnp.int32)
        qx_ref[...] = jnp.where(iota_sel == 0, lx0, 0.0)
        qy_ref[...] = jnp.where(iota_sel == 0, ly0, 0.0)
        qz_ref[...] = jnp.where(iota_sel == 0, lz0, 0.0)
        dists0 = jnp.full((SR, LC), jnp.inf, jnp.float32)

        def step(i, carry):
            lx, ly, lz, dists = carry
            dx, dy, dz = px - lx, py - ly, pz - lz
            d = (dx * dx + dy * dy) + dz * dz
            dists = jnp.minimum(dists, d)
            m = jnp.max(dists)
            idx = jnp.min(jnp.where(dists == m, iota_src, jnp.int32(2**30)))
            hit = iota_src == idx
            nlx = jnp.max(jnp.where(hit, px, -1.0))
            nly = jnp.max(jnp.where(hit, py, -1.0))
            nlz = jnp.max(jnp.where(hit, pz, -1.0))
            put = iota_sel == i
            sel_ref[...] = jnp.where(put, idx, sel_ref[...])
            qx_ref[...] = jnp.where(put, nlx, qx_ref[...])
            qy_ref[...] = jnp.where(put, nly, qy_ref[...])
            qz_ref[...] = jnp.where(put, nlz, qz_ref[...])
            return (nlx, nly, nlz, dists)

        lax.fori_loop(1, n_samples, step, (lx0, ly0, lz0, dists0))

    px = pos[:, 0].reshape(SR, LC)
    py = pos[:, 1].reshape(SR, LC)
    pz = pos[:, 2].reshape(SR, LC)
    sel, qx, qy, qz = pl.pallas_call(
        body,
        out_shape=(jax.ShapeDtypeStruct((OR, LC), jnp.int32),
                   jax.ShapeDtypeStruct((OR, LC), jnp.float32),
                   jax.ShapeDtypeStruct((OR, LC), jnp.float32),
                   jax.ShapeDtypeStruct((OR, LC), jnp.float32)),
    )(px, py, pz)
    pos_sel = jnp.stack([qx.reshape(-1), qy.reshape(-1), qz.reshape(-1)], axis=1)
    return sel.reshape(-1), pos_sel


def _d2_pallas(pos_x, pos_y, r2_s):
    Q, N = pos_y.shape[0], pos_x.shape[0]
    BQ = min(256, Q)

    NBLK = N // 128

    def body(y_ref, xt_ref, sy_ref, sx_ref, tri_ref, o_ref, stop_ref, r2_s):
        t = jnp.dot(y_ref[...], xt_ref[...])
        d2 = (sy_ref[...].reshape(BQ, 1)
              + sx_ref[...].reshape(1, N)) - 2.0 * t
        o_ref[...] = d2
        mask = (d2 <= r2_s).astype(jnp.float32)
        cblk = jnp.sum(mask.reshape(BQ, NBLK, 128), axis=2)
        cum = jnp.dot(cblk, tri_ref[...],
                      precision=jax.lax.Precision.HIGHEST)
        stop_ref[...] = jnp.minimum(
            jnp.sum((cum < float(_K)).astype(jnp.int32), axis=1) + 1, NBLK)

    sy = jnp.sum(pos_y ** 2, axis=1)
    sx = jnp.sum(pos_x ** 2, axis=1)
    tri = jnp.triu(jnp.ones((NBLK, NBLK), jnp.float32))
    d2, stop = pl.pallas_call(
        functools.partial(body, r2_s=r2_s),
        grid=(Q // BQ,),
        in_specs=[
            pl.BlockSpec((BQ, 3), lambda i: (i, 0)),
            pl.BlockSpec((3, N), lambda i: (0, 0)),
            pl.BlockSpec((BQ,), lambda i: (i,)),
            pl.BlockSpec((N,), lambda i: (0,)),
            pl.BlockSpec((NBLK, NBLK), lambda i: (0, 0)),
        ],
        out_specs=(pl.BlockSpec((BQ, N), lambda i: (i, 0)),
                   pl.BlockSpec((BQ,), lambda i: (i,))),
        out_shape=(jax.ShapeDtypeStruct((Q, N), jnp.float32),
                   jax.ShapeDtypeStruct((Q,), jnp.int32)),
    )(pos_y, pos_x.T, sy, sx, tri)
    return d2, stop


def _select_gather_sc(d2_flat, stop, table, r2, Q, N, F):
    NW = 32
    QW = Q // NW
    NB = N // 16
    mesh = plsc.VectorSubcoreMesh(core_axis_name="c", subcore_axis_name="s")

    @functools.partial(
        pl.kernel, mesh=mesh,
        compiler_params=pltpu.CompilerParams(
            needs_layout_passes=False, use_tc_tiling_on_sc=False),
        out_type=(jax.ShapeDtypeStruct((Q * _K, F), jnp.float32),
                  jax.ShapeDtypeStruct((Q,), jnp.int32)),
        scratch_types=[
            pltpu.VMEM((N,), jnp.float32),
            pltpu.VMEM((QW * _SLACK,), jnp.int32),
            pltpu.VMEM((_K, F), jnp.float32),
            pltpu.VMEM((QW,), jnp.int32),
            pltpu.VMEM((QW,), jnp.int32),
            pltpu.SemaphoreType.DMA,
        ],
    )
    def k(d2_hbm, stop_hbm, tab_hbm, g_hbm, cnt_hbm, row_v, nbr_v, rows_v,
          cnt_v, stop_v, sem):
        wid = lax.axis_index("s") * 2 + lax.axis_index("c")
        base = wid * QW
        lane = lax.iota(jnp.int32, 16)
        lane0 = lane == 0
        pltpu.sync_copy(stop_hbm.at[pl.ds(base, QW)], stop_v.at[pl.ds(0, QW)])

        def per_query(ql, carry):
            q = base + ql
            pltpu.sync_copy(d2_hbm.at[pl.ds(q * N, N)], row_v)
            for off in range(0, _SLACK, 16):
                nbr_v[pl.ds(ql * _SLACK + off, 16)] = jnp.zeros((16,), jnp.int32)

            def scan(jb, cnt):
                dv = row_v[pl.ds(jb * 16, 16)]
                m = dv <= r2
                jv = lane + jb * 16
                rank = jnp.cumsum(m.astype(jnp.int32))
                slot = cnt + rank - 1
                sel = jnp.logical_and(m, slot < _K)
                plsc.store_scatter(nbr_v, [ql * _SLACK + slot], jv, mask=sel)
                pc = plsc.all_reduce_population_count(m)
                return cnt + pc[0]

            nchunks = stop_v[pl.ds(ql, 16)][0] * (128 // 16)
            cnt = lax.fori_loop(0, nchunks, scan, jnp.int32(0))
            cntq = jnp.minimum(cnt, jnp.int32(_K))
            plsc.store_scatter(cnt_v, [jnp.broadcast_to(ql, (16,))],
                               jnp.broadcast_to(cntq, (16,)), mask=lane0)
            pltpu.async_copy(
                tab_hbm.at[nbr_v.at[pl.ds(ql * _SLACK, _K)]], rows_v, sem
            ).wait()
            pltpu.sync_copy(rows_v, g_hbm.at[pl.ds(q * _K, _K)])
            return carry

        lax.fori_loop(0, QW, per_query, 0)
        pltpu.sync_copy(cnt_v, cnt_hbm.at[pl.ds(base, QW)])

    return k(d2_flat, stop, table)


def _conv_pallas(G3, posrep, vmask, xdim, W1, b1, W2, b2):
    Q, _, F = G3.shape
    H = W1.shape[1]
    C = W2.shape[1]
    BQ = min(128, Q)

    def body(g_ref, pr_ref, vm_ref, w1_ref, b1_ref, w2_ref, b2_ref, o_ref):
        g2 = g_ref[...].reshape(BQ * _K, F)
        p2 = pr_ref[...].reshape(BQ * _K, 4)
        rel = g2[:, xdim:xdim + 3] - p2[:, :3]
        cat = jnp.concatenate([g2[:, :xdim], rel], axis=1)
        h = jnp.maximum(jnp.dot(cat, w1_ref[...]) + b1_ref[...], 0.0)
        msg = jnp.dot(h, w2_ref[...]) + b2_ref[...]
        m3 = msg.reshape(BQ, _K, C)
        m3 = jnp.where(vm_ref[...] > 0, m3, -jnp.inf)
        o_ref[...] = jnp.max(m3, axis=1)

    return pl.pallas_call(
        body,
        grid=(Q // BQ,),
        in_specs=[
            pl.BlockSpec((BQ, _K, F), lambda i: (i, 0, 0)),
            pl.BlockSpec((BQ, _K, 4), lambda i: (i, 0, 0)),
            pl.BlockSpec((BQ, _K, 1), lambda i: (i, 0, 0)),
            pl.BlockSpec((xdim + 3, H), lambda i: (0, 0)),
            pl.BlockSpec((1, H), lambda i: (0, 0)),
            pl.BlockSpec((H, C), lambda i: (0, 0)),
            pl.BlockSpec((1, C), lambda i: (0, 0)),
        ],
        out_specs=pl.BlockSpec((BQ, C), lambda i: (i, 0)),
        out_shape=jax.ShapeDtypeStruct((Q, C), jnp.float32),
    )(G3, posrep, vmask, W1, b1, W2, b2)


def _sa_stage(pos_src, x_src, pos_dst, r, W1, b1, W2, b2):
    N = pos_src.shape[0]
    Q = pos_dst.shape[0]
    xdim = 3 if x_src is None else x_src.shape[1]
    F = ((xdim + 3 + 15) // 16) * 16

    parts = [pos_src if x_src is None else x_src, pos_src]
    table = jnp.concatenate(parts, axis=1)
    table = jnp.pad(table, ((0, 0), (0, F - (xdim + 3))))

    r2 = np.float32(r * r)
    d2, stop = _d2_pallas(pos_src, pos_dst, r2)
    G, cnt = _select_gather_sc(d2.reshape(-1), stop, table, r2, Q, N, F)
    G3 = G.reshape(Q, _K, F)

    posrep = jnp.broadcast_to(
        jnp.pad(pos_dst, ((0, 0), (0, 1)))[:, None, :], (Q, _K, 4))
    vmask = (jnp.arange(_K, dtype=jnp.int32)[None, :]
             < cnt[:, None]).astype(jnp.float32)[:, :, None]
    return _conv_pallas(G3, posrep, vmask, xdim, W1,
                        b1.reshape(1, -1), W2, b2.reshape(1, -1))


def kernel(pos, W11, b11, W12, b12, W21, b21, W22, b22, W31, b31, W32, b32,
           batch):
    n1 = pos.shape[0] // 2
    idx1, pos1 = _fps_pallas(pos, n1)
    x1 = _sa_stage(pos, None, pos1, 0.2, W11, b11, W12, b12)
    n2 = n1 // 4
    idx2, pos2 = _fps_pallas(pos1, n2)
    x2 = _sa_stage(pos1, x1, pos2, 0.4, W21, b21, W22, b22)
    x3 = _sa_stage(pos2, x2, pos2, 1.0, W31, b31, W32, b32)
    batch3 = jnp.take(jnp.take(batch, idx1), idx2)
    return (x3, pos2, batch3)
